# Initial kernel scaffold; baseline (speedup 1.0000x reference)
import jax
import jax.numpy as jnp
from jax.experimental import pallas as pl


def kernel(edge_index, X, u_Y, W1, b1, W2, b2):
    raise NotImplementedError("write your pallas kernel here")


# speedup vs baseline: 20.0272x; 20.0272x over previous
import functools

import jax
import jax.numpy as jnp
from jax import lax
from jax.experimental import pallas as pl
from jax.experimental.pallas import tpu as pltpu
from jax.experimental.pallas import tpu_sc as plsc

N = 10000
E = 320000
D_IN = 128
HID = 512
NPAD = 10240
DUMMY = N
NC = 2
NS = 16
CHUNK = 128
K = 79
EPAD = NC * NS * K * CHUNK
ROWB = 512
NBLK = NPAD // ROWB
TILE_ROWS = NPAD // NS

_mesh = plsc.VectorSubcoreMesh(core_axis_name="c", subcore_axis_name="s")
_sc_params = pltpu.CompilerParams(needs_layout_passes=False)


@functools.partial(
    pl.kernel,
    out_type=jax.ShapeDtypeStruct((NC * NS, NPAD), jnp.float32),
    mesh=_mesh,
    compiler_params=_sc_params,
    scratch_types=[
        pltpu.VMEM((NPAD,), jnp.float32),
        pltpu.VMEM((CHUNK,), jnp.int32),
    ],
)
def _sc_degree(dst_hbm, out_hbm, hist, dbuf):
    cid = lax.axis_index("c")
    sid = lax.axis_index("s")
    wid = cid * NS + sid
    zeros16 = jnp.zeros((16,), jnp.float32)
    ones16 = jnp.ones((16,), jnp.float32)

    def zero_body(i, _):
        hist[pl.ds(i * 16, 16)] = zeros16
        return 0

    lax.fori_loop(0, NPAD // 16, zero_body, 0)

    def chunk_body(j, _):
        pltpu.sync_copy(dst_hbm.at[cid, sid, j], dbuf)
        for i in range(CHUNK // 16):
            idx = dbuf[pl.ds(i * 16, 16)]
            plsc.addupdate_scatter(hist, [idx], ones16)
        return 0

    lax.fori_loop(0, K, chunk_body, 0)
    pltpu.sync_copy(hist, out_hbm.at[wid])


@functools.partial(
    pl.kernel,
    out_type=jax.ShapeDtypeStruct((NC, NPAD, D_IN), jnp.float32),
    mesh=_mesh,
    compiler_params=_sc_params,
    scratch_types=[
        pltpu.VMEM_SHARED((NPAD, D_IN), jnp.float32),
        pltpu.VMEM((CHUNK, D_IN), jnp.float32),
        pltpu.VMEM((CHUNK,), jnp.int32),
        pltpu.VMEM((CHUNK,), jnp.int32),
        pltpu.SemaphoreType.DMA,
    ],
)
def _sc_seg128(src_hbm, dst_hbm, a_hbm, out_hbm, acc, rows, sbuf, dbuf, sem):
    cid = lax.axis_index("c")
    sid = lax.axis_index("s")
    zeros16 = jnp.zeros((16,), jnp.float32)

    def zero_rows(i, _):
        r = i // (D_IN // 16)
        col = i % (D_IN // 16)
        rows[r, pl.ds(col * 16, 16)] = zeros16
        return 0

    lax.fori_loop(0, CHUNK * (D_IN // 16), zero_rows, 0)
    base = sid * TILE_ROWS
    for k in range(TILE_ROWS // CHUNK):
        pltpu.sync_copy(rows, acc.at[pl.ds(base + k * CHUNK, CHUNK)])
    plsc.subcore_barrier()

    def chunk_body(j, _):
        pltpu.sync_copy(src_hbm.at[cid, sid, j], sbuf)
        pltpu.sync_copy(dst_hbm.at[cid, sid, j], dbuf)
        pltpu.async_copy(a_hbm.at[sbuf], rows, sem).wait()
        pltpu.sync_copy(rows, acc.at[dbuf], add=True)
        return 0

    lax.fori_loop(0, K, chunk_body, 0)
    plsc.subcore_barrier()
    pltpu.sync_copy(acc.at[pl.ds(base, TILE_ROWS)],
                    out_hbm.at[cid, pl.ds(base, TILE_ROWS)])


@functools.partial(
    pl.kernel,
    out_type=jax.ShapeDtypeStruct((NC * NS, NPAD), jnp.float32),
    mesh=_mesh,
    compiler_params=_sc_params,
    scratch_types=[
        pltpu.VMEM((NPAD,), jnp.float32),
        pltpu.VMEM((NPAD,), jnp.float32),
        pltpu.VMEM((CHUNK,), jnp.int32),
        pltpu.VMEM((CHUNK,), jnp.int32),
    ],
)
def _sc_segscalar(src_hbm, dst_hbm, c_hbm, out_hbm, cloc, tloc, sbuf, dbuf):
    cid = lax.axis_index("c")
    sid = lax.axis_index("s")
    wid = cid * NS + sid
    zeros16 = jnp.zeros((16,), jnp.float32)
    pltpu.sync_copy(c_hbm, cloc)

    def zero_body(i, _):
        tloc[pl.ds(i * 16, 16)] = zeros16
        return 0

    lax.fori_loop(0, NPAD // 16, zero_body, 0)

    def chunk_body(j, _):
        pltpu.sync_copy(src_hbm.at[cid, sid, j], sbuf)
        pltpu.sync_copy(dst_hbm.at[cid, sid, j], dbuf)
        for i in range(CHUNK // 16):
            sidx = sbuf[pl.ds(i * 16, 16)]
            didx = dbuf[pl.ds(i * 16, 16)]
            v = plsc.load_gather(cloc, [sidx])
            plsc.addupdate_scatter(tloc, [didx], v)
        return 0

    lax.fori_loop(0, K, chunk_body, 0)
    pltpu.sync_copy(tloc, out_hbm.at[wid])


def _tc_prep_body(latent_ref, degp_ref, a_ref, dinv_ref):
    deg = 1.0 + jnp.sum(degp_ref[...], axis=0)
    dinv = lax.rsqrt(deg)
    a_ref[...] = latent_ref[...] * dinv[:, None]
    dinv_ref[...] = dinv[:, None]


def _tc_prep(latent_pad, deg_parts):
    return pl.pallas_call(
        _tc_prep_body,
        grid=(NBLK,),
        in_specs=[
            pl.BlockSpec((ROWB, D_IN), lambda i: (i, 0)),
            pl.BlockSpec((NC * NS, ROWB), lambda i: (0, i)),
        ],
        out_specs=[
            pl.BlockSpec((ROWB, D_IN), lambda i: (i, 0)),
            pl.BlockSpec((ROWB, 1), lambda i: (i, 0)),
        ],
        out_shape=[
            jax.ShapeDtypeStruct((NPAD, D_IN), jnp.float32),
            jax.ShapeDtypeStruct((NPAD, 1), jnp.float32),
        ],
    )(latent_pad, deg_parts)


def _tc_mlp_body(parts_ref, a_ref, dinv_ref, w1_ref, b1_ref, w2_ref, c_ref):
    i = pl.program_id(0)
    dinv = dinv_ref[...]
    z = (parts_ref[0] + parts_ref[1] + a_ref[...]) * dinv
    h = jnp.maximum(
        jnp.dot(z, w1_ref[...], precision=lax.Precision.HIGHEST,
                preferred_element_type=jnp.float32) + b1_ref[...],
        0.0,
    )
    w2d = w2_ref[:, 0:1] - w2_ref[:, 1:2]
    c = jnp.dot(h, w2d, precision=lax.Precision.HIGHEST,
                preferred_element_type=jnp.float32) * dinv
    row = i * ROWB + lax.broadcasted_iota(jnp.int32, (ROWB, 1), 0)
    c_ref[...] = jnp.where(row < N, c, 0.0)


def _tc_mlp(parts, a, dinv, w1, b1, w2):
    return pl.pallas_call(
        _tc_mlp_body,
        grid=(NBLK,),
        in_specs=[
            pl.BlockSpec((NC, ROWB, D_IN), lambda i: (0, i, 0)),
            pl.BlockSpec((ROWB, D_IN), lambda i: (i, 0)),
            pl.BlockSpec((ROWB, 1), lambda i: (i, 0)),
            pl.BlockSpec((D_IN, HID), lambda i: (0, 0)),
            pl.BlockSpec((1, HID), lambda i: (0, 0)),
            pl.BlockSpec((HID, 2), lambda i: (0, 0)),
        ],
        out_specs=pl.BlockSpec((ROWB, 1), lambda i: (i, 0)),
        out_shape=jax.ShapeDtypeStruct((NPAD, 1), jnp.float32),
    )(parts, a, dinv, w1, b1, w2)


def _tc_finish_body(tp_ref, c_ref, dinv_ref, b2_ref, out_ref):
    t = jnp.sum(tp_ref[...], axis=0)[:, None]
    delta = dinv_ref[...] * (t + c_ref[...]) + (b2_ref[0, 0] - b2_ref[0, 1])
    pos = delta >= 0.0
    ez = jnp.exp(jnp.where(pos, -delta, delta))
    p0 = jnp.where(pos, 1.0 / (1.0 + ez), ez / (1.0 + ez))
    out_ref[...] = jnp.concatenate([p0, 1.0 - p0], axis=1)


def _tc_finish(t_parts, c, dinv, b2):
    return pl.pallas_call(
        _tc_finish_body,
        grid=(NBLK,),
        in_specs=[
            pl.BlockSpec((NC * NS, ROWB), lambda i: (0, i)),
            pl.BlockSpec((ROWB, 1), lambda i: (i, 0)),
            pl.BlockSpec((ROWB, 1), lambda i: (i, 0)),
            pl.BlockSpec((1, 2), lambda i: (0, 0)),
        ],
        out_specs=pl.BlockSpec((ROWB, 2), lambda i: (i, 0)),
        out_shape=jax.ShapeDtypeStruct((NPAD, 2), jnp.float32),
    )(t_parts, c, dinv, b2)


@jax.jit
def kernel(edge_index, X, u_Y, W1, b1, W2, b2):
    pad = jnp.full((EPAD - E,), DUMMY, jnp.int32)
    src = jnp.concatenate([edge_index[0], pad]).reshape(NC, NS, K, CHUNK)
    dst = jnp.concatenate([edge_index[1], pad]).reshape(NC, NS, K, CHUNK)
    latent_pad = jnp.zeros((NPAD, D_IN), jnp.float32)
    latent_pad = latent_pad.at[:N].set(jnp.concatenate([u_Y, X], axis=1))

    deg_parts = _sc_degree(dst)
    a, dinv = _tc_prep(latent_pad, deg_parts)
    parts = _sc_seg128(src, dst, a)
    c = _tc_mlp(parts, a, dinv, W1, b1.reshape(1, HID), W2)
    t_parts = _sc_segscalar(src, dst, c.reshape(NPAD))
    out = _tc_finish(t_parts, c, dinv, b2.reshape(1, 2))
    return out[:N]


# speedup vs baseline: 22.2387x; 1.1104x over previous
import functools

import jax
import jax.numpy as jnp
from jax import lax
from jax.experimental import pallas as pl
from jax.experimental.pallas import tpu as pltpu
from jax.experimental.pallas import tpu_sc as plsc

N = 10000
E = 320000
D_IN = 128
HID = 512
NPAD = 10240
DUMMY = N
NC = 2
NS = 16
CHUNK = 128
K = 80
EPAD = NC * NS * K * CHUNK
ROWB = 512
NBLK = NPAD // ROWB
TILE_ROWS = NPAD // NS

_mesh = plsc.VectorSubcoreMesh(core_axis_name="c", subcore_axis_name="s")
_sc_params = pltpu.CompilerParams(needs_layout_passes=False)


@functools.partial(
    pl.kernel,
    out_type=jax.ShapeDtypeStruct((NC * NS, NPAD), jnp.float32),
    mesh=_mesh,
    compiler_params=_sc_params,
    scratch_types=[
        pltpu.VMEM((NPAD,), jnp.float32),
        pltpu.VMEM((K, CHUNK), jnp.int32),
    ],
)
def _sc_degree(dst_hbm, out_hbm, hist, dall):
    cid = lax.axis_index("c")
    sid = lax.axis_index("s")
    wid = cid * NS + sid
    zeros16 = jnp.zeros((16,), jnp.float32)
    ones16 = jnp.ones((16,), jnp.float32)
    pltpu.sync_copy(dst_hbm.at[cid, sid], dall)

    def zero_body(i, _):
        hist[pl.ds(i * 16, 16)] = zeros16
        return 0

    lax.fori_loop(0, NPAD // 16, zero_body, 0)

    def chunk_body(j, _):
        for i in range(CHUNK // 16):
            idx = dall[j, pl.ds(i * 16, 16)]
            plsc.addupdate_scatter(hist, [idx], ones16)
        return 0

    lax.fori_loop(0, K, chunk_body, 0)
    pltpu.sync_copy(hist, out_hbm.at[wid])


@functools.partial(
    pl.kernel,
    out_type=jax.ShapeDtypeStruct((NC, NPAD, D_IN), jnp.float32),
    mesh=_mesh,
    compiler_params=_sc_params,
    scratch_types=[
        pltpu.VMEM_SHARED((NPAD, D_IN), jnp.float32),
        pltpu.VMEM((CHUNK, D_IN), jnp.float32),
        pltpu.VMEM((CHUNK, D_IN), jnp.float32),
        pltpu.VMEM((4, CHUNK), jnp.int32),
        pltpu.VMEM((K, CHUNK), jnp.int32),
        pltpu.SemaphoreType.DMA,
        pltpu.SemaphoreType.DMA,
        pltpu.SemaphoreType.DMA,
        pltpu.SemaphoreType.DMA,
        pltpu.SemaphoreType.DMA,
        pltpu.SemaphoreType.DMA,
        pltpu.SemaphoreType.DMA,
        pltpu.SemaphoreType.DMA,
    ],
)
def _sc_seg128(src_hbm, dst_hbm, a_hbm, out_hbm, acc, rows0, rows1,
               sring, dall, gs0, gs1, ss0, ss1, is0, is1, is2, is3):
    cid = lax.axis_index("c")
    sid = lax.axis_index("s")
    zeros16 = jnp.zeros((16,), jnp.float32)
    pltpu.sync_copy(dst_hbm.at[cid, sid], dall)

    def zero_rows(i, _):
        r = i // (D_IN // 16)
        col = i % (D_IN // 16)
        rows0[r, pl.ds(col * 16, 16)] = zeros16
        return 0

    lax.fori_loop(0, CHUNK * (D_IN // 16), zero_rows, 0)
    base = sid * TILE_ROWS
    for k in range(TILE_ROWS // CHUNK):
        pltpu.sync_copy(rows0, acc.at[pl.ds(base + k * CHUNK, CHUNK)])
    plsc.subcore_barrier()

    isems = [is0, is1, is2, is3]

    def idx_start(jj, slot):
        pltpu.async_copy(src_hbm.at[cid, sid, jj], sring.at[slot], isems[slot])

    def idx_wait(jj, slot):
        pltpu.make_async_copy(src_hbm.at[cid, sid, jj], sring.at[slot],
                              isems[slot]).wait()

    def gat_start(slot, rows, gsem):
        pltpu.async_copy(a_hbm.at[sring.at[slot]], rows, gsem)

    def gat_wait(slot, rows, gsem):
        pltpu.make_async_copy(a_hbm.at[sring.at[slot]], rows, gsem).wait()

    def sca_start(jj, rows, ssem):
        pltpu.async_copy(rows, acc.at[dall.at[jj]], ssem, add=True)

    def sca_wait(jj, rows, ssem):
        pltpu.make_async_copy(rows, acc.at[dall.at[jj]], ssem).wait()

    pltpu.sync_copy(src_hbm.at[cid, sid, 0], sring.at[0])
    pltpu.sync_copy(src_hbm.at[cid, sid, 1], sring.at[1])
    gat_start(0, rows0, gs0)
    gat_start(1, rows1, gs1)
    idx_start(2, 2)
    idx_start(3, 3)

    def pipe_body(it, _):
        j = 4 * it
        gat_wait(0, rows0, gs0)
        idx_start(j + 4, 0)
        sca_start(j, rows0, ss0)
        gat_wait(1, rows1, gs1)
        idx_start(j + 5, 1)
        sca_start(j + 1, rows1, ss1)
        sca_wait(j, rows0, ss0)
        idx_wait(j + 2, 2)
        gat_start(2, rows0, gs0)
        sca_wait(j + 1, rows1, ss1)
        idx_wait(j + 3, 3)
        gat_start(3, rows1, gs1)
        gat_wait(2, rows0, gs0)
        idx_start(j + 6, 2)
        sca_start(j + 2, rows0, ss0)
        gat_wait(3, rows1, gs1)
        idx_start(j + 7, 3)
        sca_start(j + 3, rows1, ss1)
        sca_wait(j + 2, rows0, ss0)
        idx_wait(j + 4, 0)
        gat_start(0, rows0, gs0)
        sca_wait(j + 3, rows1, ss1)
        idx_wait(j + 5, 1)
        gat_start(1, rows1, gs1)
        return 0

    lax.fori_loop(0, (K - 4) // 4, pipe_body, 0)

    jl = K - 4
    gat_wait(0, rows0, gs0)
    sca_start(jl, rows0, ss0)
    gat_wait(1, rows1, gs1)
    sca_start(jl + 1, rows1, ss1)
    sca_wait(jl, rows0, ss0)
    idx_wait(jl + 2, 2)
    gat_start(2, rows0, gs0)
    sca_wait(jl + 1, rows1, ss1)
    idx_wait(jl + 3, 3)
    gat_start(3, rows1, gs1)
    gat_wait(2, rows0, gs0)
    sca_start(jl + 2, rows0, ss0)
    gat_wait(3, rows1, gs1)
    sca_start(jl + 3, rows1, ss1)
    sca_wait(jl + 2, rows0, ss0)
    sca_wait(jl + 3, rows1, ss1)

    plsc.subcore_barrier()
    pltpu.sync_copy(acc.at[pl.ds(base, TILE_ROWS)],
                    out_hbm.at[cid, pl.ds(base, TILE_ROWS)])


@functools.partial(
    pl.kernel,
    out_type=jax.ShapeDtypeStruct((NC * NS, NPAD), jnp.float32),
    mesh=_mesh,
    compiler_params=_sc_params,
    scratch_types=[
        pltpu.VMEM((NPAD,), jnp.float32),
        pltpu.VMEM((NPAD,), jnp.float32),
        pltpu.VMEM((K, CHUNK), jnp.int32),
        pltpu.VMEM((K, CHUNK), jnp.int32),
    ],
)
def _sc_segscalar(src_hbm, dst_hbm, c_hbm, out_hbm, cloc, tloc, sall, dall):
    cid = lax.axis_index("c")
    sid = lax.axis_index("s")
    wid = cid * NS + sid
    zeros16 = jnp.zeros((16,), jnp.float32)
    pltpu.sync_copy(src_hbm.at[cid, sid], sall)
    pltpu.sync_copy(dst_hbm.at[cid, sid], dall)
    pltpu.sync_copy(c_hbm, cloc)

    def zero_body(i, _):
        tloc[pl.ds(i * 16, 16)] = zeros16
        return 0

    lax.fori_loop(0, NPAD // 16, zero_body, 0)

    def chunk_body(j, _):
        for i in range(CHUNK // 16):
            sidx = sall[j, pl.ds(i * 16, 16)]
            didx = dall[j, pl.ds(i * 16, 16)]
            v = plsc.load_gather(cloc, [sidx])
            plsc.addupdate_scatter(tloc, [didx], v)
        return 0

    lax.fori_loop(0, K, chunk_body, 0)
    pltpu.sync_copy(tloc, out_hbm.at[wid])


def _tc_prep_body(latent_ref, degp_ref, a_ref, dinv_ref):
    deg = 1.0 + jnp.sum(degp_ref[...], axis=0)
    dinv = lax.rsqrt(deg)
    a_ref[...] = latent_ref[...] * dinv[:, None]
    dinv_ref[...] = dinv[:, None]


def _tc_prep(latent_pad, deg_parts):
    return pl.pallas_call(
        _tc_prep_body,
        grid=(NBLK,),
        in_specs=[
            pl.BlockSpec((ROWB, D_IN), lambda i: (i, 0)),
            pl.BlockSpec((NC * NS, ROWB), lambda i: (0, i)),
        ],
        out_specs=[
            pl.BlockSpec((ROWB, D_IN), lambda i: (i, 0)),
            pl.BlockSpec((ROWB, 1), lambda i: (i, 0)),
        ],
        out_shape=[
            jax.ShapeDtypeStruct((NPAD, D_IN), jnp.float32),
            jax.ShapeDtypeStruct((NPAD, 1), jnp.float32),
        ],
    )(latent_pad, deg_parts)


def _tc_mlp_body(parts_ref, a_ref, dinv_ref, w1_ref, b1_ref, w2_ref, c_ref):
    i = pl.program_id(0)
    dinv = dinv_ref[...]
    z = (parts_ref[0] + parts_ref[1] + a_ref[...]) * dinv
    h = jnp.maximum(
        jnp.dot(z, w1_ref[...], precision=lax.Precision.HIGHEST,
                preferred_element_type=jnp.float32) + b1_ref[...],
        0.0,
    )
    w2d = w2_ref[:, 0:1] - w2_ref[:, 1:2]
    c = jnp.dot(h, w2d, precision=lax.Precision.HIGHEST,
                preferred_element_type=jnp.float32) * dinv
    row = i * ROWB + lax.broadcasted_iota(jnp.int32, (ROWB, 1), 0)
    c_ref[...] = jnp.where(row < N, c, 0.0)


def _tc_mlp(parts, a, dinv, w1, b1, w2):
    return pl.pallas_call(
        _tc_mlp_body,
        grid=(NBLK,),
        in_specs=[
            pl.BlockSpec((NC, ROWB, D_IN), lambda i: (0, i, 0)),
            pl.BlockSpec((ROWB, D_IN), lambda i: (i, 0)),
            pl.BlockSpec((ROWB, 1), lambda i: (i, 0)),
            pl.BlockSpec((D_IN, HID), lambda i: (0, 0)),
            pl.BlockSpec((1, HID), lambda i: (0, 0)),
            pl.BlockSpec((HID, 2), lambda i: (0, 0)),
        ],
        out_specs=pl.BlockSpec((ROWB, 1), lambda i: (i, 0)),
        out_shape=jax.ShapeDtypeStruct((NPAD, 1), jnp.float32),
    )(parts, a, dinv, w1, b1, w2)


def _tc_finish_body(tp_ref, c_ref, dinv_ref, b2_ref, out_ref):
    t = jnp.sum(tp_ref[...], axis=0)[:, None]
    delta = dinv_ref[...] * (t + c_ref[...]) + (b2_ref[0, 0] - b2_ref[0, 1])
    pos = delta >= 0.0
    ez = jnp.exp(jnp.where(pos, -delta, delta))
    p0 = jnp.where(pos, 1.0 / (1.0 + ez), ez / (1.0 + ez))
    out_ref[...] = jnp.concatenate([p0, 1.0 - p0], axis=1)


def _tc_finish(t_parts, c, dinv, b2):
    return pl.pallas_call(
        _tc_finish_body,
        grid=(NBLK,),
        in_specs=[
            pl.BlockSpec((NC * NS, ROWB), lambda i: (0, i)),
            pl.BlockSpec((ROWB, 1), lambda i: (i, 0)),
            pl.BlockSpec((ROWB, 1), lambda i: (i, 0)),
            pl.BlockSpec((1, 2), lambda i: (0, 0)),
        ],
        out_specs=pl.BlockSpec((ROWB, 2), lambda i: (i, 0)),
        out_shape=jax.ShapeDtypeStruct((NPAD, 2), jnp.float32),
    )(t_parts, c, dinv, b2)


@jax.jit
def kernel(edge_index, X, u_Y, W1, b1, W2, b2):
    pad = jnp.full((EPAD - E,), DUMMY, jnp.int32)
    src = jnp.concatenate([edge_index[0], pad]).reshape(NC, NS, K, CHUNK)
    dst = jnp.concatenate([edge_index[1], pad]).reshape(NC, NS, K, CHUNK)
    latent_pad = jnp.zeros((NPAD, D_IN), jnp.float32)
    latent_pad = latent_pad.at[:N].set(jnp.concatenate([u_Y, X], axis=1))

    deg_parts = _sc_degree(dst)
    a, dinv = _tc_prep(latent_pad, deg_parts)
    parts = _sc_seg128(src, dst, a)
    c = _tc_mlp(parts, a, dinv, W1, b1.reshape(1, HID), W2)
    t_parts = _sc_segscalar(src, dst, c.reshape(NPAD))
    out = _tc_finish(t_parts, c, dinv, b2.reshape(1, 2))
    return out[:N]


# speedup vs baseline: 41.1037x; 1.8483x over previous
import functools

import jax
import jax.numpy as jnp
from jax import lax
from jax.experimental import pallas as pl
from jax.experimental.pallas import tpu as pltpu
from jax.experimental.pallas import tpu_sc as plsc

N = 10000
E = 320000
D_IN = 128
HID = 512
NPAD = 10240
NC = 2
NS = 16
CHUNK = 128
K = 80
EPAD = NC * NS * K * CHUNK
ROWB = 512
NBLK = NPAD // ROWB
TILE_ROWS = NPAD // NS

_mesh = plsc.VectorSubcoreMesh(core_axis_name="c", subcore_axis_name="s")
_sc_params = pltpu.CompilerParams(needs_layout_passes=False)


@functools.partial(
    pl.kernel,
    out_type=jax.ShapeDtypeStruct((NC * NS, NPAD), jnp.float32),
    mesh=_mesh,
    compiler_params=_sc_params,
    scratch_types=[
        pltpu.VMEM((NPAD,), jnp.float32),
        pltpu.VMEM((K, CHUNK), jnp.int32),
    ],
)
def _sc_degree(dst_hbm, out_hbm, hist, dall):
    cid = lax.axis_index("c")
    sid = lax.axis_index("s")
    wid = cid * NS + sid
    zeros16 = jnp.zeros((16,), jnp.float32)
    ones16 = jnp.ones((16,), jnp.float32)
    pltpu.sync_copy(dst_hbm.at[cid, sid], dall)

    def zero_body(i, _):
        hist[pl.ds(i * 16, 16)] = zeros16
        return 0

    lax.fori_loop(0, NPAD // 16, zero_body, 0)

    def chunk_body(j, _):
        for i in range(CHUNK // 16):
            idx = dall[j, pl.ds(i * 16, 16)]
            plsc.addupdate_scatter(hist, [idx], ones16)
        return 0

    lax.fori_loop(0, K, chunk_body, 0)
    pltpu.sync_copy(hist, out_hbm.at[wid])


@functools.partial(
    pl.kernel,
    out_type=jax.ShapeDtypeStruct((NC, NPAD, D_IN), jnp.float32),
    mesh=_mesh,
    compiler_params=_sc_params,
    scratch_types=[
        pltpu.VMEM_SHARED((NPAD, D_IN), jnp.float32),
        pltpu.VMEM((CHUNK, D_IN), jnp.float32),
        pltpu.VMEM((CHUNK, D_IN), jnp.float32),
        pltpu.VMEM((4, CHUNK), jnp.int32),
        pltpu.VMEM((K, CHUNK), jnp.int32),
        pltpu.SemaphoreType.DMA,
        pltpu.SemaphoreType.DMA,
        pltpu.SemaphoreType.DMA,
        pltpu.SemaphoreType.DMA,
        pltpu.SemaphoreType.DMA,
        pltpu.SemaphoreType.DMA,
        pltpu.SemaphoreType.DMA,
        pltpu.SemaphoreType.DMA,
    ],
)
def _sc_seg128(src_hbm, dst_hbm, a_hbm, out_hbm, acc, rows0, rows1,
               sring, dall, gs0, gs1, ss0, ss1, is0, is1, is2, is3):
    cid = lax.axis_index("c")
    sid = lax.axis_index("s")
    zeros16 = jnp.zeros((16,), jnp.float32)
    pltpu.sync_copy(dst_hbm.at[cid, sid], dall)

    def zero_rows(i, _):
        r = i // (D_IN // 16)
        col = i % (D_IN // 16)
        rows0[r, pl.ds(col * 16, 16)] = zeros16
        return 0

    lax.fori_loop(0, CHUNK * (D_IN // 16), zero_rows, 0)
    base = sid * TILE_ROWS
    for k in range(TILE_ROWS // CHUNK):
        pltpu.sync_copy(rows0, acc.at[pl.ds(base + k * CHUNK, CHUNK)])
    plsc.subcore_barrier()

    isems = [is0, is1, is2, is3]

    def idx_start(jj, slot):
        pltpu.async_copy(src_hbm.at[cid, sid, jj], sring.at[slot], isems[slot])

    def idx_wait(jj, slot):
        pltpu.make_async_copy(src_hbm.at[cid, sid, jj], sring.at[slot],
                              isems[slot]).wait()

    def gat_start(slot, rows, gsem):
        pltpu.async_copy(a_hbm.at[sring.at[slot]], rows, gsem)

    def gat_wait(slot, rows, gsem):
        pltpu.make_async_copy(a_hbm.at[sring.at[slot]], rows, gsem).wait()

    def sca_start(jj, rows, ssem):
        pltpu.async_copy(rows, acc.at[dall.at[jj]], ssem, add=True)

    def sca_wait(jj, rows, ssem):
        pltpu.make_async_copy(rows, acc.at[dall.at[jj]], ssem).wait()

    pltpu.sync_copy(src_hbm.at[cid, sid, 0], sring.at[0])
    pltpu.sync_copy(src_hbm.at[cid, sid, 1], sring.at[1])
    gat_start(0, rows0, gs0)
    gat_start(1, rows1, gs1)
    idx_start(2, 2)
    idx_start(3, 3)

    def pipe_body(it, _):
        j = 4 * it
        gat_wait(0, rows0, gs0)
        idx_start(j + 4, 0)
        sca_start(j, rows0, ss0)
        gat_wait(1, rows1, gs1)
        idx_start(j + 5, 1)
        sca_start(j + 1, rows1, ss1)
        sca_wait(j, rows0, ss0)
        idx_wait(j + 2, 2)
        gat_start(2, rows0, gs0)
        sca_wait(j + 1, rows1, ss1)
        idx_wait(j + 3, 3)
        gat_start(3, rows1, gs1)
        gat_wait(2, rows0, gs0)
        idx_start(j + 6, 2)
        sca_start(j + 2, rows0, ss0)
        gat_wait(3, rows1, gs1)
        idx_start(j + 7, 3)
        sca_start(j + 3, rows1, ss1)
        sca_wait(j + 2, rows0, ss0)
        idx_wait(j + 4, 0)
        gat_start(0, rows0, gs0)
        sca_wait(j + 3, rows1, ss1)
        idx_wait(j + 5, 1)
        gat_start(1, rows1, gs1)
        return 0

    lax.fori_loop(0, (K - 4) // 4, pipe_body, 0)

    jl = K - 4
    gat_wait(0, rows0, gs0)
    sca_start(jl, rows0, ss0)
    gat_wait(1, rows1, gs1)
    sca_start(jl + 1, rows1, ss1)
    sca_wait(jl, rows0, ss0)
    idx_wait(jl + 2, 2)
    gat_start(2, rows0, gs0)
    sca_wait(jl + 1, rows1, ss1)
    idx_wait(jl + 3, 3)
    gat_start(3, rows1, gs1)
    gat_wait(2, rows0, gs0)
    sca_start(jl + 2, rows0, ss0)
    gat_wait(3, rows1, gs1)
    sca_start(jl + 3, rows1, ss1)
    sca_wait(jl + 2, rows0, ss0)
    sca_wait(jl + 3, rows1, ss1)

    plsc.subcore_barrier()
    pltpu.sync_copy(acc.at[pl.ds(base, TILE_ROWS)],
                    out_hbm.at[cid, pl.ds(base, TILE_ROWS)])


@functools.partial(
    pl.kernel,
    out_type=jax.ShapeDtypeStruct((NC * NS, NPAD), jnp.float32),
    mesh=_mesh,
    compiler_params=_sc_params,
    scratch_types=[
        pltpu.VMEM((NPAD,), jnp.float32),
        pltpu.VMEM((NPAD,), jnp.float32),
        pltpu.VMEM((K, CHUNK), jnp.int32),
        pltpu.VMEM((K, CHUNK), jnp.int32),
    ],
)
def _sc_segscalar(src_hbm, dst_hbm, c_hbm, out_hbm, cloc, tloc, sall, dall):
    cid = lax.axis_index("c")
    sid = lax.axis_index("s")
    wid = cid * NS + sid
    zeros16 = jnp.zeros((16,), jnp.float32)
    pltpu.sync_copy(src_hbm.at[cid, sid], sall)
    pltpu.sync_copy(dst_hbm.at[cid, sid], dall)
    pltpu.sync_copy(c_hbm, cloc)

    def zero_body(i, _):
        tloc[pl.ds(i * 16, 16)] = zeros16
        return 0

    lax.fori_loop(0, NPAD // 16, zero_body, 0)

    def chunk_body(j, _):
        for i in range(CHUNK // 16):
            sidx = sall[j, pl.ds(i * 16, 16)]
            didx = dall[j, pl.ds(i * 16, 16)]
            v = plsc.load_gather(cloc, [sidx])
            plsc.addupdate_scatter(tloc, [didx], v)
        return 0

    lax.fori_loop(0, K, chunk_body, 0)
    pltpu.sync_copy(tloc, out_hbm.at[wid])


def _tc_prep_body(latent_ref, degp_ref, a_ref, dinv_ref):
    deg = 1.0 + jnp.sum(degp_ref[...], axis=0)
    dinv = lax.rsqrt(deg)
    a_ref[...] = latent_ref[...] * dinv[:, None]
    dinv_ref[...] = dinv[:, None]


def _tc_prep(latent_pad, deg_parts):
    return pl.pallas_call(
        _tc_prep_body,
        grid=(NBLK,),
        in_specs=[
            pl.BlockSpec((ROWB, D_IN), lambda i: (i, 0)),
            pl.BlockSpec((NC * NS, ROWB), lambda i: (0, i)),
        ],
        out_specs=[
            pl.BlockSpec((ROWB, D_IN), lambda i: (i, 0)),
            pl.BlockSpec((ROWB, 1), lambda i: (i, 0)),
        ],
        out_shape=[
            jax.ShapeDtypeStruct((NPAD, D_IN), jnp.float32),
            jax.ShapeDtypeStruct((NPAD, 1), jnp.float32),
        ],
    )(latent_pad, deg_parts)


def _tc_mlp_body(parts_ref, a_ref, dinv_ref, w1_ref, b1_ref, w2_ref, c_ref):
    i = pl.program_id(0)
    dinv = dinv_ref[...]
    z = (parts_ref[0] + parts_ref[1] + a_ref[...]) * dinv
    h = jnp.maximum(
        jnp.dot(z, w1_ref[...], precision=lax.Precision.HIGHEST,
                preferred_element_type=jnp.float32) + b1_ref[...],
        0.0,
    )
    w2d = w2_ref[:, 0:1] - w2_ref[:, 1:2]
    c = jnp.dot(h, w2d, precision=lax.Precision.HIGHEST,
                preferred_element_type=jnp.float32) * dinv
    row = i * ROWB + lax.broadcasted_iota(jnp.int32, (ROWB, 1), 0)
    c_ref[...] = jnp.where(row < N, c, 0.0)


def _tc_mlp(parts, a, dinv, w1, b1, w2):
    return pl.pallas_call(
        _tc_mlp_body,
        grid=(NBLK,),
        in_specs=[
            pl.BlockSpec((NC, ROWB, D_IN), lambda i: (0, i, 0)),
            pl.BlockSpec((ROWB, D_IN), lambda i: (i, 0)),
            pl.BlockSpec((ROWB, 1), lambda i: (i, 0)),
            pl.BlockSpec((D_IN, HID), lambda i: (0, 0)),
            pl.BlockSpec((1, HID), lambda i: (0, 0)),
            pl.BlockSpec((HID, 2), lambda i: (0, 0)),
        ],
        out_specs=pl.BlockSpec((ROWB, 1), lambda i: (i, 0)),
        out_shape=jax.ShapeDtypeStruct((NPAD, 1), jnp.float32),
    )(parts, a, dinv, w1, b1, w2)


def _tc_finish_body(tp_ref, c_ref, dinv_ref, b2_ref, out_ref):
    t = jnp.sum(tp_ref[...], axis=0)[:, None]
    delta = dinv_ref[...] * (t + c_ref[...]) + (b2_ref[0, 0] - b2_ref[0, 1])
    pos = delta >= 0.0
    ez = jnp.exp(jnp.where(pos, -delta, delta))
    p0 = jnp.where(pos, 1.0 / (1.0 + ez), ez / (1.0 + ez))
    out_ref[...] = jnp.concatenate([p0, 1.0 - p0], axis=1)


def _tc_finish(t_parts, c, dinv, b2):
    return pl.pallas_call(
        _tc_finish_body,
        grid=(NBLK,),
        in_specs=[
            pl.BlockSpec((NC * NS, ROWB), lambda i: (0, i)),
            pl.BlockSpec((ROWB, 1), lambda i: (i, 0)),
            pl.BlockSpec((ROWB, 1), lambda i: (i, 0)),
            pl.BlockSpec((1, 2), lambda i: (0, 0)),
        ],
        out_specs=pl.BlockSpec((ROWB, 2), lambda i: (i, 0)),
        out_shape=jax.ShapeDtypeStruct((NPAD, 2), jnp.float32),
    )(t_parts, c, dinv, b2)


@jax.jit
def kernel(edge_index, X, u_Y, W1, b1, W2, b2):
    pad = N + jnp.arange(EPAD - E, dtype=jnp.int32) % (NPAD - N)
    src = jnp.concatenate([edge_index[0], pad]).reshape(NC, NS, K, CHUNK)
    dst = jnp.concatenate([edge_index[1], pad]).reshape(NC, NS, K, CHUNK)
    latent_pad = jnp.zeros((NPAD, D_IN), jnp.float32)
    latent_pad = latent_pad.at[:N].set(jnp.concatenate([u_Y, X], axis=1))

    deg_parts = _sc_degree(dst)
    a, dinv = _tc_prep(latent_pad, deg_parts)
    parts = _sc_seg128(src, dst, a)
    c = _tc_mlp(parts, a, dinv, W1, b1.reshape(1, HID), W2)
    t_parts = _sc_segscalar(src, dst, c.reshape(NPAD))
    out = _tc_finish(t_parts, c, dinv, b2.reshape(1, 2))
    return out[:N]


# speedup vs baseline: 47.9760x; 1.1672x over previous
import functools

import jax
import jax.numpy as jnp
from jax import lax
from jax.experimental import pallas as pl
from jax.experimental.pallas import tpu as pltpu
from jax.experimental.pallas import tpu_sc as plsc

N = 10000
E = 320000
D_IN = 128
HID = 512
NPAD = 10240
NC = 2
NS = 16
CHUNK = 128
K = 80
EPAD = NC * NS * K * CHUNK
ROWB = 512
NBLK = NPAD // ROWB
TILE_ROWS = NPAD // NS

_mesh = plsc.VectorSubcoreMesh(core_axis_name="c", subcore_axis_name="s")
_sc_params = pltpu.CompilerParams(needs_layout_passes=False)


@functools.partial(
    pl.kernel,
    out_type=jax.ShapeDtypeStruct((NC * NS, NPAD), jnp.float32),
    mesh=_mesh,
    compiler_params=_sc_params,
    scratch_types=[
        pltpu.VMEM((NPAD,), jnp.float32),
        pltpu.VMEM((K, CHUNK), jnp.int32),
    ],
)
def _sc_degree(dst_hbm, out_hbm, hist, dall):
    cid = lax.axis_index("c")
    sid = lax.axis_index("s")
    wid = cid * NS + sid
    zeros16 = jnp.zeros((16,), jnp.float32)
    ones16 = jnp.ones((16,), jnp.float32)
    pltpu.sync_copy(dst_hbm.at[cid, sid], dall)

    def zero_body(i, _):
        hist[pl.ds(i * 16, 16)] = zeros16
        return 0

    lax.fori_loop(0, NPAD // 16, zero_body, 0)

    def chunk_body(j, _):
        for i in range(CHUNK // 16):
            idx = dall[j, pl.ds(i * 16, 16)]
            plsc.addupdate_scatter(hist, [idx], ones16)
        return 0

    lax.fori_loop(0, K, chunk_body, 0)
    pltpu.sync_copy(hist, out_hbm.at[wid])


@functools.partial(
    pl.kernel,
    out_type=jax.ShapeDtypeStruct((NC, NPAD, D_IN), jnp.float32),
    mesh=_mesh,
    compiler_params=_sc_params,
    scratch_types=[
        pltpu.VMEM_SHARED((NPAD, D_IN), jnp.float32),
        pltpu.VMEM((CHUNK, D_IN), jnp.float32),
        pltpu.VMEM((CHUNK, D_IN), jnp.float32),
        pltpu.VMEM((4, CHUNK), jnp.int32),
        pltpu.VMEM((K, CHUNK), jnp.int32),
        pltpu.SemaphoreType.DMA,
        pltpu.SemaphoreType.DMA,
        pltpu.SemaphoreType.DMA,
        pltpu.SemaphoreType.DMA,
        pltpu.SemaphoreType.DMA,
        pltpu.SemaphoreType.DMA,
        pltpu.SemaphoreType.DMA,
        pltpu.SemaphoreType.DMA,
    ],
)
def _sc_seg128(src_hbm, dst_hbm, a_hbm, out_hbm, acc, rows0, rows1,
               sring, dall, gs0, gs1, ss0, ss1, is0, is1, is2, is3):
    cid = lax.axis_index("c")
    sid = lax.axis_index("s")
    zeros16 = jnp.zeros((16,), jnp.float32)
    pltpu.sync_copy(dst_hbm.at[cid, sid], dall)

    def zero_rows(i, _):
        r = i // (D_IN // 16)
        col = i % (D_IN // 16)
        rows0[r, pl.ds(col * 16, 16)] = zeros16
        return 0

    lax.fori_loop(0, CHUNK * (D_IN // 16), zero_rows, 0)
    base = sid * TILE_ROWS
    for k in range(TILE_ROWS // CHUNK):
        pltpu.sync_copy(rows0, acc.at[pl.ds(base + k * CHUNK, CHUNK)])
    plsc.subcore_barrier()

    isems = [is0, is1, is2, is3]

    def idx_start(jj, slot):
        pltpu.async_copy(src_hbm.at[cid, sid, jj], sring.at[slot], isems[slot])

    def idx_wait(jj, slot):
        pltpu.make_async_copy(src_hbm.at[cid, sid, jj], sring.at[slot],
                              isems[slot]).wait()

    def gat_start(slot, rows, gsem):
        pltpu.async_copy(a_hbm.at[sring.at[slot]], rows, gsem)

    def gat_wait(slot, rows, gsem):
        pltpu.make_async_copy(a_hbm.at[sring.at[slot]], rows, gsem).wait()

    def sca_start(jj, rows, ssem):
        pltpu.async_copy(rows, acc.at[dall.at[jj]], ssem, add=True)

    def sca_wait(jj, rows, ssem):
        pltpu.make_async_copy(rows, acc.at[dall.at[jj]], ssem).wait()

    pltpu.sync_copy(src_hbm.at[cid, sid, 0], sring.at[0])
    pltpu.sync_copy(src_hbm.at[cid, sid, 1], sring.at[1])
    gat_start(0, rows0, gs0)
    gat_start(1, rows1, gs1)
    idx_start(2, 2)
    idx_start(3, 3)

    def pipe_body(it, _):
        j = 4 * it
        gat_wait(0, rows0, gs0)
        idx_start(j + 4, 0)
        sca_start(j, rows0, ss0)
        gat_wait(1, rows1, gs1)
        idx_start(j + 5, 1)
        sca_start(j + 1, rows1, ss1)
        sca_wait(j, rows0, ss0)
        idx_wait(j + 2, 2)
        gat_start(2, rows0, gs0)
        sca_wait(j + 1, rows1, ss1)
        idx_wait(j + 3, 3)
        gat_start(3, rows1, gs1)
        gat_wait(2, rows0, gs0)
        idx_start(j + 6, 2)
        sca_start(j + 2, rows0, ss0)
        gat_wait(3, rows1, gs1)
        idx_start(j + 7, 3)
        sca_start(j + 3, rows1, ss1)
        sca_wait(j + 2, rows0, ss0)
        idx_wait(j + 4, 0)
        gat_start(0, rows0, gs0)
        sca_wait(j + 3, rows1, ss1)
        idx_wait(j + 5, 1)
        gat_start(1, rows1, gs1)
        return 0

    lax.fori_loop(0, (K - 4) // 4, pipe_body, 0)

    jl = K - 4
    gat_wait(0, rows0, gs0)
    sca_start(jl, rows0, ss0)
    gat_wait(1, rows1, gs1)
    sca_start(jl + 1, rows1, ss1)
    sca_wait(jl, rows0, ss0)
    idx_wait(jl + 2, 2)
    gat_start(2, rows0, gs0)
    sca_wait(jl + 1, rows1, ss1)
    idx_wait(jl + 3, 3)
    gat_start(3, rows1, gs1)
    gat_wait(2, rows0, gs0)
    sca_start(jl + 2, rows0, ss0)
    gat_wait(3, rows1, gs1)
    sca_start(jl + 3, rows1, ss1)
    sca_wait(jl + 2, rows0, ss0)
    sca_wait(jl + 3, rows1, ss1)

    plsc.subcore_barrier()
    pltpu.sync_copy(acc.at[pl.ds(base, TILE_ROWS)],
                    out_hbm.at[cid, pl.ds(base, TILE_ROWS)])


@functools.partial(
    pl.kernel,
    out_type=jax.ShapeDtypeStruct((NC * NS, NPAD), jnp.float32),
    mesh=_mesh,
    compiler_params=_sc_params,
    scratch_types=[
        pltpu.VMEM((NPAD,), jnp.float32),
        pltpu.VMEM((NPAD,), jnp.float32),
        pltpu.VMEM((K, CHUNK), jnp.int32),
        pltpu.VMEM((K, CHUNK), jnp.int32),
    ],
)
def _sc_segscalar(src_hbm, dst_hbm, c_hbm, out_hbm, cloc, tloc, sall, dall):
    cid = lax.axis_index("c")
    sid = lax.axis_index("s")
    wid = cid * NS + sid
    zeros16 = jnp.zeros((16,), jnp.float32)
    pltpu.sync_copy(src_hbm.at[cid, sid], sall)
    pltpu.sync_copy(dst_hbm.at[cid, sid], dall)
    pltpu.sync_copy(c_hbm, cloc)

    def zero_body(i, _):
        tloc[pl.ds(i * 16, 16)] = zeros16
        return 0

    lax.fori_loop(0, NPAD // 16, zero_body, 0)

    def chunk_body(j, _):
        for i in range(CHUNK // 16):
            sidx = sall[j, pl.ds(i * 16, 16)]
            didx = dall[j, pl.ds(i * 16, 16)]
            v = plsc.load_gather(cloc, [sidx])
            plsc.addupdate_scatter(tloc, [didx], v)
        return 0

    lax.fori_loop(0, K, chunk_body, 0)
    pltpu.sync_copy(tloc, out_hbm.at[wid])


def _tc_prep_body(uy_ref, x_ref, degp_ref, a_ref, dinv_ref):
    deg = 1.0 + jnp.sum(degp_ref[...], axis=0)
    dinv = lax.rsqrt(deg)[:, None]
    latent = jnp.concatenate([uy_ref[...], x_ref[...]], axis=1)
    latent = jnp.concatenate(
        [latent, jnp.zeros((NPAD - N, D_IN), jnp.float32)], axis=0)
    a_ref[...] = latent * dinv
    dinv_ref[...] = dinv


def _tc_prep(u_Y, X, deg_parts):
    return pl.pallas_call(
        _tc_prep_body,
        out_shape=[
            jax.ShapeDtypeStruct((NPAD, D_IN), jnp.float32),
            jax.ShapeDtypeStruct((NPAD, 1), jnp.float32),
        ],
    )(u_Y, X, deg_parts)


def _tc_mlp_body(parts_ref, a_ref, dinv_ref, w1_ref, b1_ref, w2_ref, c_ref):
    i = pl.program_id(0)
    dinv = dinv_ref[...]
    z = (parts_ref[0] + parts_ref[1] + a_ref[...]) * dinv
    h = jnp.maximum(
        jnp.dot(z, w1_ref[...], preferred_element_type=jnp.float32)
        + b1_ref[...],
        0.0,
    )
    w2d = w2_ref[:, 0:1] - w2_ref[:, 1:2]
    c = jnp.dot(h, w2d, preferred_element_type=jnp.float32) * dinv
    row = i * ROWB + lax.broadcasted_iota(jnp.int32, (ROWB, 1), 0)
    c_ref[...] = jnp.where(row < N, c, 0.0)


def _tc_mlp(parts, a, dinv, w1, b1, w2):
    return pl.pallas_call(
        _tc_mlp_body,
        grid=(NBLK,),
        in_specs=[
            pl.BlockSpec((NC, ROWB, D_IN), lambda i: (0, i, 0)),
            pl.BlockSpec((ROWB, D_IN), lambda i: (i, 0)),
            pl.BlockSpec((ROWB, 1), lambda i: (i, 0)),
            pl.BlockSpec((D_IN, HID), lambda i: (0, 0)),
            pl.BlockSpec((1, HID), lambda i: (0, 0)),
            pl.BlockSpec((HID, 2), lambda i: (0, 0)),
        ],
        out_specs=pl.BlockSpec((ROWB, 1), lambda i: (i, 0)),
        out_shape=jax.ShapeDtypeStruct((NPAD, 1), jnp.float32),
    )(parts, a, dinv, w1, b1, w2)


def _tc_finish_body(tp_ref, c_ref, dinv_ref, b2_ref, out_ref):
    t = jnp.sum(tp_ref[...], axis=0)[:, None]
    delta = dinv_ref[...] * (t + c_ref[...]) + (b2_ref[0, 0] - b2_ref[0, 1])
    pos = delta >= 0.0
    ez = jnp.exp(jnp.where(pos, -delta, delta))
    p0 = jnp.where(pos, 1.0 / (1.0 + ez), ez / (1.0 + ez))
    out_ref[...] = jnp.concatenate([p0, 1.0 - p0], axis=1)


def _tc_finish(t_parts, c, dinv, b2):
    return pl.pallas_call(
        _tc_finish_body,
        grid=(NBLK,),
        in_specs=[
            pl.BlockSpec((NC * NS, ROWB), lambda i: (0, i)),
            pl.BlockSpec((ROWB, 1), lambda i: (i, 0)),
            pl.BlockSpec((ROWB, 1), lambda i: (i, 0)),
            pl.BlockSpec((1, 2), lambda i: (0, 0)),
        ],
        out_specs=pl.BlockSpec((ROWB, 2), lambda i: (i, 0)),
        out_shape=jax.ShapeDtypeStruct((NPAD, 2), jnp.float32),
    )(t_parts, c, dinv, b2)


@jax.jit
def kernel(edge_index, X, u_Y, W1, b1, W2, b2):
    pad = N + jnp.arange(EPAD - E, dtype=jnp.int32) % (NPAD - N)
    src = jnp.concatenate([edge_index[0], pad]).reshape(NC, NS, K, CHUNK)
    dst = jnp.concatenate([edge_index[1], pad]).reshape(NC, NS, K, CHUNK)
    deg_parts = _sc_degree(dst)
    a, dinv = _tc_prep(u_Y, X, deg_parts)
    parts = _sc_seg128(src, dst, a)
    c = _tc_mlp(parts, a, dinv, W1, b1.reshape(1, HID), W2)
    t_parts = _sc_segscalar(src, dst, c.reshape(NPAD))
    out = _tc_finish(t_parts, c, dinv, b2.reshape(1, 2))
    return out[:N]
